# Initial kernel scaffold; baseline (speedup 1.0000x reference)
import jax
import jax.numpy as jnp
from jax.experimental import pallas as pl


def kernel(x, edge_index, W1, b1, W2, b2, W3, b3):
    raise NotImplementedError("write your pallas kernel here")


# speedup vs baseline: 11.4354x; 11.4354x over previous
import functools

import jax
import jax.numpy as jnp
from jax import lax
from jax.experimental import pallas as pl
from jax.experimental.pallas import tpu as pltpu
from jax.experimental.pallas import tpu_sc as plsc

N = 50000
NPAD = 50176
E = 800000
EPAD = 819200
IDX_ROWS = EPAD // 128
NC, NS = 2, 16
TILE_ROWS = NPAD // NS
CH = 8
F32 = jnp.float32


def _sc_mesh():
    return plsc.VectorSubcoreMesh(core_axis_name="c", subcore_axis_name="s")


_SC_PARAMS = pltpu.CompilerParams(use_tc_tiling_on_sc=False)


def _deg_call(dst2d, zeros_col, ones_rows):
    rows_per_tile = IDX_ROWS // (NC * NS)
    n_chunks = rows_per_tile // CH

    @functools.partial(
        pl.kernel,
        out_type=jax.ShapeDtypeStruct((NC, NPAD, 1), F32),
        mesh=_sc_mesh(),
        compiler_params=_SC_PARAMS,
        scratch_types=[
            pltpu.VMEM((CH, 128), jnp.int32),
            pltpu.VMEM((128, 1), F32),
            pltpu.VMEM((TILE_ROWS, 1), F32),
            pltpu.VMEM_SHARED((NPAD, 1), F32),
            pltpu.SemaphoreType.DMA,
        ],
    )
    def deg_kernel(dst_hbm, zero_hbm, ones_hbm, out_hbm,
                   dst_v, ones_v, stage_v, acc, sem):
        cid = lax.axis_index("c")
        sid = lax.axis_index("s")
        wid = cid * NS + sid
        base = sid * TILE_ROWS
        pltpu.sync_copy(zero_hbm.at[pl.ds(base, TILE_ROWS)], stage_v)
        pltpu.sync_copy(stage_v, acc.at[pl.ds(base, TILE_ROWS)])
        pltpu.sync_copy(ones_hbm, ones_v)
        plsc.subcore_barrier()

        row0 = wid * rows_per_tile

        def body(i, carry):
            r = row0 + i * CH
            pltpu.sync_copy(dst_hbm.at[pl.ds(r, CH)], dst_v)
            for j in range(CH):
                pltpu.sync_copy(ones_v, acc.at[dst_v.at[j]], add=True)
            return carry

        lax.fori_loop(0, n_chunks, body, 0)
        plsc.subcore_barrier()
        pltpu.sync_copy(acc.at[pl.ds(base, TILE_ROWS)], stage_v)
        pltpu.sync_copy(stage_v, out_hbm.at[cid].at[pl.ds(base, TILE_ROWS)])

    return deg_kernel(dst2d, zeros_col, ones_rows)


def _agg_call(table_pair, src2d, dst2d, H, ch):
    rows_per_tile = IDX_ROWS // NS
    n_chunks = rows_per_tile // ch
    chunk_e = ch * 128
    init_chunk = 784 if chunk_e >= 784 else 392
    n_init = TILE_ROWS // init_chunk

    @functools.partial(
        pl.kernel,
        out_type=jax.ShapeDtypeStruct((NC, NPAD, H), F32),
        mesh=_sc_mesh(),
        compiler_params=_SC_PARAMS,
        scratch_types=[
            pltpu.VMEM((ch, 128), jnp.int32),
            pltpu.VMEM((ch, 128), jnp.int32),
            pltpu.VMEM((chunk_e, H), F32),
            pltpu.VMEM_SHARED((NPAD, H), F32),
            pltpu.SemaphoreType.DMA,
        ],
    )
    def agg_kernel(table_hbm, src_hbm, dst_hbm, out_hbm,
                   src_v, dst_v, rows_v, acc, sem):
        cid = lax.axis_index("c")
        sid = lax.axis_index("s")
        base = sid * TILE_ROWS

        for i in range(n_init):
            r = base + i * init_chunk
            pltpu.sync_copy(table_hbm.at[cid].at[pl.ds(r, init_chunk)],
                            rows_v.at[pl.ds(0, init_chunk)])
            pltpu.sync_copy(rows_v.at[pl.ds(0, init_chunk)],
                            acc.at[pl.ds(r, init_chunk)])
        plsc.subcore_barrier()

        row0 = sid * rows_per_tile

        def body(i, carry):
            r = row0 + i * ch
            pltpu.sync_copy(src_hbm.at[pl.ds(r, ch)], src_v)
            pltpu.sync_copy(dst_hbm.at[pl.ds(r, ch)], dst_v)
            for j in range(ch):
                pltpu.async_copy(table_hbm.at[cid].at[src_v.at[j]],
                                 rows_v.at[pl.ds(j * 128, 128)], sem).wait()
            for j in range(ch):
                pltpu.sync_copy(rows_v.at[pl.ds(j * 128, 128)],
                                acc.at[dst_v.at[j]], add=True)
            return carry

        lax.fori_loop(0, n_chunks, body, 0)
        plsc.subcore_barrier()

        for i in range(n_init):
            r = base + i * init_chunk
            pltpu.sync_copy(acc.at[pl.ds(r, init_chunk)],
                            rows_v.at[pl.ds(0, init_chunk)])
            pltpu.sync_copy(rows_v.at[pl.ds(0, init_chunk)],
                            out_hbm.at[cid].at[pl.ds(r, init_chunk)])

    return agg_kernel(table_pair, src2d, dst2d)


BR = 512
GRID = (NPAD // BR,)


def _row_spec(w):
    return pl.BlockSpec((BR, w), lambda i: (i, 0))


def _pair_spec(w):
    return pl.BlockSpec((2, BR, w), lambda i: (0, i, 0))


def _full_spec(a, b):
    return pl.BlockSpec((a, b), lambda i: (0, 0))


def _tc_prep(x_pad, deg0, deg1, W1):
    def body(x_ref, d0_ref, d1_ref, w_ref, dinv_ref, out_ref):
        deg = d0_ref[...] + d1_ref[...] + 1.0
        dv = lax.rsqrt(deg)
        h = jnp.dot(x_ref[...], w_ref[...], preferred_element_type=F32)
        hs = h * dv
        dinv_ref[...] = dv
        out_ref[0] = hs[:, :16]
        out_ref[1] = hs[:, 16:]

    return pl.pallas_call(
        body,
        grid=GRID,
        in_specs=[_row_spec(64), _row_spec(1), _row_spec(1), _full_spec(64, 32)],
        out_specs=[_row_spec(1), _pair_spec(16)],
        out_shape=[
            jax.ShapeDtypeStruct((NPAD, 1), F32),
            jax.ShapeDtypeStruct((2, NPAD, 16), F32),
        ],
    )(x_pad, deg0, deg1, W1)


def _tc_mid(agg, dinv, bias, W, Hin, Hout):
    def body(a_ref, dv_ref, b_ref, w_ref, out_ref):
        dv = dv_ref[...]
        cat = jnp.concatenate([a_ref[0], a_ref[1]], axis=1)
        z = jnp.maximum(dv * cat + b_ref[...], 0.0)
        h = jnp.dot(z, w_ref[...], preferred_element_type=F32)
        hs = h * dv
        out_ref[0] = hs[:, :Hout // 2]
        out_ref[1] = hs[:, Hout // 2:]

    return pl.pallas_call(
        body,
        grid=GRID,
        in_specs=[_pair_spec(Hin // 2), _row_spec(1), _full_spec(1, Hin),
                  _full_spec(Hin, Hout)],
        out_specs=[_pair_spec(Hout // 2)],
        out_shape=[jax.ShapeDtypeStruct((2, NPAD, Hout // 2), F32)],
    )(agg, dinv, bias, W)[0]


def _tc_final(agg, dinv, bias):
    def body(a_ref, dv_ref, b_ref, out_ref):
        cat = jnp.concatenate([a_ref[0], a_ref[1]], axis=1)
        emb = dv_ref[...] * cat + b_ref[...]
        nrm = jnp.sqrt(jnp.sum(emb * emb, axis=-1, keepdims=True))
        out_ref[...] = emb / jnp.maximum(nrm, 1e-12)

    return pl.pallas_call(
        body,
        grid=GRID,
        in_specs=[_pair_spec(32), _row_spec(1), _full_spec(1, 64)],
        out_specs=pl.BlockSpec((BR, 64), lambda i: (i, 0)),
        out_shape=jax.ShapeDtypeStruct((NPAD, 64), F32),
    )(agg, dinv, bias)


def kernel(x, edge_index, W1, b1, W2, b2, W3, b3):
    src = edge_index[0].astype(jnp.int32)
    dst = edge_index[1].astype(jnp.int32)
    pad_e = EPAD - E
    src2d = jnp.concatenate(
        [src, jnp.zeros((pad_e,), jnp.int32)]).reshape(IDX_ROWS, 128)
    dst2d = jnp.concatenate(
        [dst, jnp.full((pad_e,), NPAD - 1, jnp.int32)]).reshape(IDX_ROWS, 128)
    x_pad = jnp.zeros((NPAD, 64), F32).at[:N].set(x)
    zeros_col = jnp.zeros((NPAD, 1), F32)
    ones_rows = jnp.ones((128, 1), F32)

    deg_parts = _deg_call(dst2d, zeros_col, ones_rows)
    dinv, hs1 = _tc_prep(x_pad, deg_parts[0], deg_parts[1], W1)
    agg1 = _agg_call(hs1, src2d, dst2d, 16, 8)
    hs2 = _tc_mid(agg1, dinv, b1.reshape(1, 32), W2, 32, 64)
    agg2 = _agg_call(hs2, src2d, dst2d, 32, 4)
    hs3 = _tc_mid(agg2, dinv, b2.reshape(1, 64), W3, 64, 64)
    agg3 = _agg_call(hs3, src2d, dst2d, 32, 4)
    emb = _tc_final(agg3, dinv, b3.reshape(1, 64))
    return emb[:N]


# speedup vs baseline: 16.1123x; 1.4090x over previous
import functools

import jax
import jax.numpy as jnp
from jax import lax
from jax.experimental import pallas as pl
from jax.experimental.pallas import tpu as pltpu
from jax.experimental.pallas import tpu_sc as plsc

N = 50000
NPAD = 50176
E = 800000
EPAD = 819200
IDX_ROWS = EPAD // 128
NC, NS = 2, 16
TILE_ROWS = NPAD // NS
F32 = jnp.float32


def _sc_mesh():
    return plsc.VectorSubcoreMesh(core_axis_name="c", subcore_axis_name="s")


_SC_PARAMS = pltpu.CompilerParams(use_tc_tiling_on_sc=False)


def _init_chunk_for(ch):
    cap = ch * 128
    for c in (784, 448, 392, 224, 112, 56):
        if c <= cap:
            return c
    raise ValueError(ch)


def _deg_call(edges3d, zeros_col, ones_rows):
    ch = 8
    rows_per_tile = IDX_ROWS // (NC * NS)
    n_chunks = rows_per_tile // ch

    @functools.partial(
        pl.kernel,
        out_type=jax.ShapeDtypeStruct((NC, NPAD, 1), F32),
        mesh=_sc_mesh(),
        compiler_params=_SC_PARAMS,
        scratch_types=[
            pltpu.VMEM((ch, 2, 128), jnp.int32),
            pltpu.VMEM((128, 1), F32),
            pltpu.VMEM((TILE_ROWS, 1), F32),
            pltpu.VMEM_SHARED((NPAD, 1), F32),
            pltpu.SemaphoreType.DMA,
        ],
    )
    def deg_kernel(e_hbm, zero_hbm, ones_hbm, out_hbm,
                   idx_v, ones_v, stage_v, acc, sem):
        cid = lax.axis_index("c")
        sid = lax.axis_index("s")
        wid = cid * NS + sid
        base = sid * TILE_ROWS
        pltpu.sync_copy(zero_hbm.at[pl.ds(base, TILE_ROWS)], stage_v)
        pltpu.sync_copy(stage_v, acc.at[pl.ds(base, TILE_ROWS)])
        pltpu.sync_copy(ones_hbm, ones_v)
        plsc.subcore_barrier()

        row0 = wid * rows_per_tile

        def body(i, carry):
            r = row0 + i * ch
            pltpu.sync_copy(e_hbm.at[pl.ds(r, ch)], idx_v)
            for j in range(ch):
                pltpu.async_copy(ones_v, acc.at[idx_v.at[j, 1]], sem,
                                 add=True)
            for j in range(ch):
                pltpu.make_async_copy(ones_v, acc.at[idx_v.at[j, 1]],
                                      sem).wait()
            return carry

        lax.fori_loop(0, n_chunks, body, 0)
        plsc.subcore_barrier()
        pltpu.sync_copy(acc.at[pl.ds(base, TILE_ROWS)], stage_v)
        pltpu.sync_copy(stage_v, out_hbm.at[cid].at[pl.ds(base, TILE_ROWS)])

    return deg_kernel(edges3d, zeros_col, ones_rows)


def _agg_call(table_pair, edges3d, H, ch):
    rows_per_tile = IDX_ROWS // NS
    n_pairs = rows_per_tile // (2 * ch)
    chunk_e = ch * 128
    init_chunk = _init_chunk_for(ch)
    n_init = TILE_ROWS // init_chunk

    @functools.partial(
        pl.kernel,
        out_type=jax.ShapeDtypeStruct((NC, NPAD, H), F32),
        mesh=_sc_mesh(),
        compiler_params=_SC_PARAMS,
        scratch_types=[
            pltpu.VMEM((ch, 2, 128), jnp.int32),
            pltpu.VMEM((ch, 2, 128), jnp.int32),
            pltpu.VMEM((chunk_e, H), F32),
            pltpu.VMEM((chunk_e, H), F32),
            pltpu.VMEM_SHARED((NPAD, H), F32),
            pltpu.SemaphoreType.DMA,
            pltpu.SemaphoreType.DMA,
            pltpu.SemaphoreType.DMA,
            pltpu.SemaphoreType.DMA,
        ],
    )
    def agg_kernel(table_hbm, e_hbm, out_hbm,
                   i0, i1, r0, r1, acc, g0, g1, s0, s1):
        cid = lax.axis_index("c")
        sid = lax.axis_index("s")
        base = sid * TILE_ROWS

        pltpu.sync_copy(table_hbm.at[cid].at[pl.ds(base, TILE_ROWS)],
                        acc.at[pl.ds(base, TILE_ROWS)])
        plsc.subcore_barrier()

        row0 = sid * rows_per_tile
        idx = (i0, i1)
        rows = (r0, r1)
        gsem = (g0, g1)
        ssem = (s0, s1)

        def fire_g(b):
            for j in range(ch):
                pltpu.async_copy(table_hbm.at[cid].at[idx[b].at[j, 0]],
                                 rows[b].at[pl.ds(j * 128, 128)], gsem[b])

        def drain_g(b):
            for j in range(ch):
                pltpu.make_async_copy(
                    table_hbm.at[cid].at[idx[b].at[j, 0]],
                    rows[b].at[pl.ds(j * 128, 128)], gsem[b]).wait()

        def fire_s(b):
            for j in range(ch):
                pltpu.async_copy(rows[b].at[pl.ds(j * 128, 128)],
                                 acc.at[idx[b].at[j, 1]], ssem[b], add=True)

        def drain_s(b):
            for j in range(ch):
                pltpu.make_async_copy(rows[b].at[pl.ds(j * 128, 128)],
                                      acc.at[idx[b].at[j, 1]], ssem[b]).wait()

        pltpu.sync_copy(e_hbm.at[pl.ds(row0, ch)], i0)
        fire_g(0)

        def body(p, carry):
            r = row0 + 2 * p * ch
            pltpu.sync_copy(e_hbm.at[pl.ds(r + ch, ch)], i1)
            drain_g(0)
            fire_s(0)
            fire_g(1)
            drain_s(0)

            @pl.when(p + 1 < n_pairs)
            def _():
                pltpu.sync_copy(e_hbm.at[pl.ds(r + 2 * ch, ch)], i0)

            drain_g(1)
            fire_s(1)

            @pl.when(p + 1 < n_pairs)
            def _():
                fire_g(0)

            drain_s(1)
            return carry

        lax.fori_loop(0, n_pairs, body, 0)
        plsc.subcore_barrier()

        pltpu.sync_copy(acc.at[pl.ds(base, TILE_ROWS)],
                        out_hbm.at[cid].at[pl.ds(base, TILE_ROWS)])

    return agg_kernel(table_pair, edges3d)


BR = 512
GRID = (NPAD // BR,)


def _row_spec(w):
    return pl.BlockSpec((BR, w), lambda i: (i, 0))


def _pair_spec(w):
    return pl.BlockSpec((2, BR, w), lambda i: (0, i, 0))


def _full_spec(a, b):
    return pl.BlockSpec((a, b), lambda i: (0, 0))


def _tc_prep(x_pad, deg0, deg1, W1):
    def body(x_ref, d0_ref, d1_ref, w_ref, dinv_ref, out_ref):
        deg = d0_ref[...] + d1_ref[...] + 1.0
        dv = lax.rsqrt(deg)
        h = jnp.dot(x_ref[...], w_ref[...], preferred_element_type=F32)
        hs = h * dv
        dinv_ref[...] = dv
        out_ref[0] = hs[:, :16]
        out_ref[1] = hs[:, 16:]

    return pl.pallas_call(
        body,
        grid=GRID,
        in_specs=[_row_spec(64), _row_spec(1), _row_spec(1), _full_spec(64, 32)],
        out_specs=[_row_spec(1), _pair_spec(16)],
        out_shape=[
            jax.ShapeDtypeStruct((NPAD, 1), F32),
            jax.ShapeDtypeStruct((2, NPAD, 16), F32),
        ],
    )(x_pad, deg0, deg1, W1)


def _tc_mid(agg, dinv, bias, W, Hin, Hout):
    def body(a_ref, dv_ref, b_ref, w_ref, out_ref):
        dv = dv_ref[...]
        cat = jnp.concatenate([a_ref[0], a_ref[1]], axis=1)
        z = jnp.maximum(dv * cat + b_ref[...], 0.0)
        h = jnp.dot(z, w_ref[...], preferred_element_type=F32)
        hs = h * dv
        out_ref[0] = hs[:, :Hout // 2]
        out_ref[1] = hs[:, Hout // 2:]

    return pl.pallas_call(
        body,
        grid=GRID,
        in_specs=[_pair_spec(Hin // 2), _row_spec(1), _full_spec(1, Hin),
                  _full_spec(Hin, Hout)],
        out_specs=[_pair_spec(Hout // 2)],
        out_shape=[jax.ShapeDtypeStruct((2, NPAD, Hout // 2), F32)],
    )(agg, dinv, bias, W)[0]


def _tc_final(agg, dinv, bias):
    def body(a_ref, dv_ref, b_ref, out_ref):
        cat = jnp.concatenate([a_ref[0], a_ref[1]], axis=1)
        emb = dv_ref[...] * cat + b_ref[...]
        nrm = jnp.sqrt(jnp.sum(emb * emb, axis=-1, keepdims=True))
        out_ref[...] = emb / jnp.maximum(nrm, 1e-12)

    return pl.pallas_call(
        body,
        grid=GRID,
        in_specs=[_pair_spec(32), _row_spec(1), _full_spec(1, 64)],
        out_specs=pl.BlockSpec((BR, 64), lambda i: (i, 0)),
        out_shape=jax.ShapeDtypeStruct((NPAD, 64), F32),
    )(agg, dinv, bias)


def kernel(x, edge_index, W1, b1, W2, b2, W3, b3):
    src = edge_index[0].astype(jnp.int32)
    dst = edge_index[1].astype(jnp.int32)
    pad_e = EPAD - E
    src2d = jnp.concatenate(
        [src, jnp.zeros((pad_e,), jnp.int32)]).reshape(IDX_ROWS, 128)
    dst2d = jnp.concatenate(
        [dst, jnp.full((pad_e,), NPAD - 1, jnp.int32)]).reshape(IDX_ROWS, 128)
    edges3d = jnp.stack([src2d, dst2d], axis=1)
    x_pad = jnp.zeros((NPAD, 64), F32).at[:N].set(x)
    zeros_col = jnp.zeros((NPAD, 1), F32)
    ones_rows = jnp.ones((128, 1), F32)

    deg_parts = _deg_call(edges3d, zeros_col, ones_rows)
    dinv, hs1 = _tc_prep(x_pad, deg_parts[0], deg_parts[1], W1)
    agg1 = _agg_call(hs1, edges3d, 16, 4)
    hs2 = _tc_mid(agg1, dinv, b1.reshape(1, 32), W2, 32, 64)
    agg2 = _agg_call(hs2, edges3d, 32, 2)
    hs3 = _tc_mid(agg2, dinv, b2.reshape(1, 64), W3, 64, 64)
    agg3 = _agg_call(hs3, edges3d, 32, 2)
    emb = _tc_final(agg3, dinv, b3.reshape(1, 64))
    return emb[:N]


# speedup vs baseline: 16.6818x; 1.0353x over previous
import functools

import jax
import jax.numpy as jnp
from jax import lax
from jax.experimental import pallas as pl
from jax.experimental.pallas import tpu as pltpu
from jax.experimental.pallas import tpu_sc as plsc

N = 50000
NPAD = 50176
E = 800000
EPAD = 819200
IDX_ROWS = EPAD // 128
NC, NS = 2, 16
TILE_ROWS = NPAD // NS
F32 = jnp.float32


def _sc_mesh():
    return plsc.VectorSubcoreMesh(core_axis_name="c", subcore_axis_name="s")


_SC_PARAMS = pltpu.CompilerParams(use_tc_tiling_on_sc=False)


def _init_chunk_for(ch):
    cap = ch * 128
    for c in (784, 448, 392, 224, 112, 56):
        if c <= cap:
            return c
    raise ValueError(ch)


def _deg_call(edges3d, zeros_col, ones_rows):
    ch = 8
    rows_per_tile = IDX_ROWS // (NC * NS)
    n_chunks = rows_per_tile // ch

    @functools.partial(
        pl.kernel,
        out_type=jax.ShapeDtypeStruct((NC, NPAD, 1), F32),
        mesh=_sc_mesh(),
        compiler_params=_SC_PARAMS,
        scratch_types=[
            pltpu.VMEM((ch, 2, 128), jnp.int32),
            pltpu.VMEM((128, 1), F32),
            pltpu.VMEM((TILE_ROWS, 1), F32),
            pltpu.VMEM_SHARED((NPAD, 1), F32),
            pltpu.SemaphoreType.DMA,
        ],
    )
    def deg_kernel(e_hbm, zero_hbm, ones_hbm, out_hbm,
                   idx_v, ones_v, stage_v, acc, sem):
        cid = lax.axis_index("c")
        sid = lax.axis_index("s")
        wid = cid * NS + sid
        base = sid * TILE_ROWS
        pltpu.sync_copy(zero_hbm.at[pl.ds(base, TILE_ROWS)], stage_v)
        pltpu.sync_copy(stage_v, acc.at[pl.ds(base, TILE_ROWS)])
        pltpu.sync_copy(ones_hbm, ones_v)
        plsc.subcore_barrier()

        row0 = wid * rows_per_tile

        def body(i, carry):
            r = row0 + i * ch
            pltpu.sync_copy(e_hbm.at[pl.ds(r, ch)], idx_v)
            for j in range(ch):
                pltpu.async_copy(ones_v, acc.at[idx_v.at[j, 1]], sem,
                                 add=True)
            for j in range(ch):
                pltpu.make_async_copy(ones_v, acc.at[idx_v.at[j, 1]],
                                      sem).wait()
            return carry

        lax.fori_loop(0, n_chunks, body, 0)
        plsc.subcore_barrier()
        pltpu.sync_copy(acc.at[pl.ds(base, TILE_ROWS)], stage_v)
        pltpu.sync_copy(stage_v, out_hbm.at[cid].at[pl.ds(base, TILE_ROWS)])

    return deg_kernel(edges3d, zeros_col, ones_rows)


def _agg_call(table_pair, edges4d, H, ch):
    rows_per_tile = IDX_ROWS // NS
    chunks_per_tile = rows_per_tile // ch
    n_pairs = chunks_per_tile // 2
    chunk_e = ch * 128

    @functools.partial(
        pl.kernel,
        out_type=jax.ShapeDtypeStruct((NC, NPAD, H), F32),
        mesh=_sc_mesh(),
        compiler_params=_SC_PARAMS,
        scratch_types=[
            pltpu.VMEM((2, chunk_e), jnp.int32),
            pltpu.VMEM((2, chunk_e), jnp.int32),
            pltpu.VMEM((chunk_e, H), F32),
            pltpu.VMEM((chunk_e, H), F32),
            pltpu.VMEM_SHARED((NPAD, H), F32),
            pltpu.SemaphoreType.DMA,
            pltpu.SemaphoreType.DMA,
            pltpu.SemaphoreType.DMA,
            pltpu.SemaphoreType.DMA,
        ],
    )
    def agg_kernel(table_hbm, e_hbm, out_hbm,
                   i0, i1, r0, r1, acc, g0, g1, s0, s1):
        cid = lax.axis_index("c")
        sid = lax.axis_index("s")
        base = sid * TILE_ROWS
        chunk0 = sid * chunks_per_tile

        pltpu.sync_copy(table_hbm.at[cid].at[pl.ds(base, TILE_ROWS)],
                        acc.at[pl.ds(base, TILE_ROWS)])
        plsc.subcore_barrier()

        idx = (i0, i1)
        rows = (r0, r1)
        gsem = (g0, g1)
        ssem = (s0, s1)

        def fire_g(b):
            pltpu.async_copy(table_hbm.at[cid].at[idx[b].at[0]],
                             rows[b], gsem[b])

        def drain_g(b):
            pltpu.make_async_copy(table_hbm.at[cid].at[idx[b].at[0]],
                                  rows[b], gsem[b]).wait()

        def fire_s(b):
            pltpu.async_copy(rows[b], acc.at[idx[b].at[1]], ssem[b],
                             add=True)

        def drain_s(b):
            pltpu.make_async_copy(rows[b], acc.at[idx[b].at[1]],
                                  ssem[b]).wait()

        pltpu.sync_copy(e_hbm.at[chunk0], i0)
        fire_g(0)

        def body(p, carry):
            g = chunk0 + 2 * p
            pltpu.sync_copy(e_hbm.at[g + 1], i1)
            drain_g(0)
            fire_s(0)
            fire_g(1)
            drain_s(0)

            @pl.when(p + 1 < n_pairs)
            def _():
                pltpu.sync_copy(e_hbm.at[g + 2], i0)

            drain_g(1)
            fire_s(1)

            @pl.when(p + 1 < n_pairs)
            def _():
                fire_g(0)

            drain_s(1)
            return carry

        lax.fori_loop(0, n_pairs, body, 0)
        plsc.subcore_barrier()

        pltpu.sync_copy(acc.at[pl.ds(base, TILE_ROWS)],
                        out_hbm.at[cid].at[pl.ds(base, TILE_ROWS)])

    return agg_kernel(table_pair, edges4d)


BR = 512
GRID = (NPAD // BR,)


def _row_spec(w):
    return pl.BlockSpec((BR, w), lambda i: (i, 0))


def _pair_spec(w):
    return pl.BlockSpec((2, BR, w), lambda i: (0, i, 0))


def _full_spec(a, b):
    return pl.BlockSpec((a, b), lambda i: (0, 0))


def _tc_prep(x_pad, deg0, deg1, W1):
    def body(x_ref, d0_ref, d1_ref, w_ref, dinv_ref, out_ref):
        deg = d0_ref[...] + d1_ref[...] + 1.0
        dv = lax.rsqrt(deg)
        h = jnp.dot(x_ref[...], w_ref[...], preferred_element_type=F32)
        hs = h * dv
        dinv_ref[...] = dv
        out_ref[0] = hs[:, :16]
        out_ref[1] = hs[:, 16:]

    return pl.pallas_call(
        body,
        grid=GRID,
        in_specs=[_row_spec(64), _row_spec(1), _row_spec(1), _full_spec(64, 32)],
        out_specs=[_row_spec(1), _pair_spec(16)],
        out_shape=[
            jax.ShapeDtypeStruct((NPAD, 1), F32),
            jax.ShapeDtypeStruct((2, NPAD, 16), F32),
        ],
    )(x_pad, deg0, deg1, W1)


def _tc_mid(agg, dinv, bias, W, Hin, Hout):
    def body(a_ref, dv_ref, b_ref, w_ref, out_ref):
        dv = dv_ref[...]
        cat = jnp.concatenate([a_ref[0], a_ref[1]], axis=1)
        z = jnp.maximum(dv * cat + b_ref[...], 0.0)
        h = jnp.dot(z, w_ref[...], preferred_element_type=F32)
        hs = h * dv
        out_ref[0] = hs[:, :Hout // 2]
        out_ref[1] = hs[:, Hout // 2:]

    return pl.pallas_call(
        body,
        grid=GRID,
        in_specs=[_pair_spec(Hin // 2), _row_spec(1), _full_spec(1, Hin),
                  _full_spec(Hin, Hout)],
        out_specs=[_pair_spec(Hout // 2)],
        out_shape=[jax.ShapeDtypeStruct((2, NPAD, Hout // 2), F32)],
    )(agg, dinv, bias, W)[0]


def _tc_final(agg, dinv, bias):
    def body(a_ref, dv_ref, b_ref, out_ref):
        cat = jnp.concatenate([a_ref[0], a_ref[1]], axis=1)
        emb = dv_ref[...] * cat + b_ref[...]
        nrm = jnp.sqrt(jnp.sum(emb * emb, axis=-1, keepdims=True))
        out_ref[...] = emb / jnp.maximum(nrm, 1e-12)

    return pl.pallas_call(
        body,
        grid=GRID,
        in_specs=[_pair_spec(32), _row_spec(1), _full_spec(1, 64)],
        out_specs=pl.BlockSpec((BR, 64), lambda i: (i, 0)),
        out_shape=jax.ShapeDtypeStruct((NPAD, 64), F32),
    )(agg, dinv, bias)


def kernel(x, edge_index, W1, b1, W2, b2, W3, b3):
    src = edge_index[0].astype(jnp.int32)
    dst = edge_index[1].astype(jnp.int32)
    pad_e = EPAD - E
    src2d = jnp.concatenate(
        [src, jnp.zeros((pad_e,), jnp.int32)]).reshape(IDX_ROWS, 128)
    dst2d = jnp.concatenate(
        [dst, jnp.full((pad_e,), NPAD - 1, jnp.int32)]).reshape(IDX_ROWS, 128)
    edges3d = jnp.stack([src2d, dst2d], axis=1)
    e4d_8 = jnp.stack([src2d.reshape(-1, 8 * 128),
                       dst2d.reshape(-1, 8 * 128)], axis=1)
    e4d_2 = jnp.stack([src2d.reshape(-1, 2 * 128),
                       dst2d.reshape(-1, 2 * 128)], axis=1)
    x_pad = jnp.zeros((NPAD, 64), F32).at[:N].set(x)
    zeros_col = jnp.zeros((NPAD, 1), F32)
    ones_rows = jnp.ones((128, 1), F32)

    deg_parts = _deg_call(edges3d, zeros_col, ones_rows)
    dinv, hs1 = _tc_prep(x_pad, deg_parts[0], deg_parts[1], W1)
    agg1 = _agg_call(hs1, e4d_8, 16, 8)
    hs2 = _tc_mid(agg1, dinv, b1.reshape(1, 32), W2, 32, 64)
    agg2 = _agg_call(hs2, e4d_2, 32, 2)
    hs3 = _tc_mid(agg2, dinv, b2.reshape(1, 64), W3, 64, 64)
    agg3 = _agg_call(hs3, e4d_2, 32, 2)
    emb = _tc_final(agg3, dinv, b3.reshape(1, 64))
    return emb[:N]


# speedup vs baseline: 16.7500x; 1.0041x over previous
import functools

import jax
import jax.numpy as jnp
from jax import lax
from jax.experimental import pallas as pl
from jax.experimental.pallas import tpu as pltpu
from jax.experimental.pallas import tpu_sc as plsc

N = 50000
NPAD = 50176
E = 800000
EPAD = 819200
IDX_ROWS = EPAD // 128
NC, NS = 2, 16
TILE_ROWS = NPAD // NS
F32 = jnp.float32


def _sc_mesh():
    return plsc.VectorSubcoreMesh(core_axis_name="c", subcore_axis_name="s")


_SC_PARAMS = pltpu.CompilerParams(use_tc_tiling_on_sc=False)


def _init_chunk_for(ch):
    cap = ch * 128
    for c in (784, 448, 392, 224, 112, 56):
        if c <= cap:
            return c
    raise ValueError(ch)


def _deg_call(edges3d, zeros_col, ones_rows):
    ch = 8
    rows_per_tile = IDX_ROWS // (NC * NS)
    n_chunks = rows_per_tile // ch

    @functools.partial(
        pl.kernel,
        out_type=jax.ShapeDtypeStruct((NC, NPAD, 1), F32),
        mesh=_sc_mesh(),
        compiler_params=_SC_PARAMS,
        scratch_types=[
            pltpu.VMEM((ch, 2, 128), jnp.int32),
            pltpu.VMEM((128, 1), F32),
            pltpu.VMEM((TILE_ROWS, 1), F32),
            pltpu.VMEM_SHARED((NPAD, 1), F32),
            pltpu.SemaphoreType.DMA,
        ],
    )
    def deg_kernel(e_hbm, zero_hbm, ones_hbm, out_hbm,
                   idx_v, ones_v, stage_v, acc, sem):
        cid = lax.axis_index("c")
        sid = lax.axis_index("s")
        wid = cid * NS + sid
        base = sid * TILE_ROWS
        pltpu.sync_copy(zero_hbm.at[pl.ds(base, TILE_ROWS)], stage_v)
        pltpu.sync_copy(stage_v, acc.at[pl.ds(base, TILE_ROWS)])
        pltpu.sync_copy(ones_hbm, ones_v)
        plsc.subcore_barrier()

        row0 = wid * rows_per_tile

        def body(i, carry):
            r = row0 + i * ch
            pltpu.sync_copy(e_hbm.at[pl.ds(r, ch)], idx_v)
            for j in range(ch):
                pltpu.async_copy(ones_v, acc.at[idx_v.at[j, 1]], sem,
                                 add=True)
            for j in range(ch):
                pltpu.make_async_copy(ones_v, acc.at[idx_v.at[j, 1]],
                                      sem).wait()
            return carry

        lax.fori_loop(0, n_chunks, body, 0)
        plsc.subcore_barrier()
        pltpu.sync_copy(acc.at[pl.ds(base, TILE_ROWS)], stage_v)
        pltpu.sync_copy(stage_v, out_hbm.at[cid].at[pl.ds(base, TILE_ROWS)])

    return deg_kernel(edges3d, zeros_col, ones_rows)


def _agg_call(table_pair, edges4d, H, chunk_e):
    edges_per_tile = EPAD // NS
    chunks_per_tile = edges_per_tile // chunk_e
    n_pairs = chunks_per_tile // 2

    @functools.partial(
        pl.kernel,
        out_type=jax.ShapeDtypeStruct((NC, NPAD, H), F32),
        mesh=_sc_mesh(),
        compiler_params=_SC_PARAMS,
        scratch_types=[
            pltpu.VMEM((2, chunk_e), jnp.int32),
            pltpu.VMEM((2, chunk_e), jnp.int32),
            pltpu.VMEM((chunk_e, H), F32),
            pltpu.VMEM((chunk_e, H), F32),
            pltpu.VMEM_SHARED((NPAD, H), F32),
            pltpu.SemaphoreType.DMA,
            pltpu.SemaphoreType.DMA,
            pltpu.SemaphoreType.DMA,
            pltpu.SemaphoreType.DMA,
        ],
    )
    def agg_kernel(table_hbm, e_hbm, out_hbm,
                   i0, i1, r0, r1, acc, g0, g1, s0, s1):
        cid = lax.axis_index("c")
        sid = lax.axis_index("s")
        base = sid * TILE_ROWS
        chunk0 = sid * chunks_per_tile

        pltpu.sync_copy(table_hbm.at[cid].at[pl.ds(base, TILE_ROWS)],
                        acc.at[pl.ds(base, TILE_ROWS)])
        plsc.subcore_barrier()

        idx = (i0, i1)
        rows = (r0, r1)
        gsem = (g0, g1)
        ssem = (s0, s1)

        def fire_g(b):
            pltpu.async_copy(table_hbm.at[cid].at[idx[b].at[0]],
                             rows[b], gsem[b])

        def drain_g(b):
            pltpu.make_async_copy(table_hbm.at[cid].at[idx[b].at[0]],
                                  rows[b], gsem[b]).wait()

        def fire_s(b):
            pltpu.async_copy(rows[b], acc.at[idx[b].at[1]], ssem[b],
                             add=True)

        def drain_s(b):
            pltpu.make_async_copy(rows[b], acc.at[idx[b].at[1]],
                                  ssem[b]).wait()

        pltpu.sync_copy(e_hbm.at[chunk0], i0)
        fire_g(0)

        def body(p, carry):
            g = chunk0 + 2 * p
            pltpu.sync_copy(e_hbm.at[g + 1], i1)
            drain_g(0)
            fire_s(0)
            fire_g(1)
            drain_s(0)

            @pl.when(p + 1 < n_pairs)
            def _():
                pltpu.sync_copy(e_hbm.at[g + 2], i0)

            drain_g(1)
            fire_s(1)

            @pl.when(p + 1 < n_pairs)
            def _():
                fire_g(0)

            drain_s(1)
            return carry

        lax.fori_loop(0, n_pairs, body, 0)
        plsc.subcore_barrier()

        pltpu.sync_copy(acc.at[pl.ds(base, TILE_ROWS)],
                        out_hbm.at[cid].at[pl.ds(base, TILE_ROWS)])

    return agg_kernel(table_pair, edges4d)


BR = 512
GRID = (NPAD // BR,)


def _row_spec(w):
    return pl.BlockSpec((BR, w), lambda i: (i, 0))


def _pair_spec(w):
    return pl.BlockSpec((2, BR, w), lambda i: (0, i, 0))


def _full_spec(a, b):
    return pl.BlockSpec((a, b), lambda i: (0, 0))


def _tc_prep(x_pad, deg0, deg1, W1):
    def body(x_ref, d0_ref, d1_ref, w_ref, dinv_ref, out_ref):
        deg = d0_ref[...] + d1_ref[...] + 1.0
        dv = lax.rsqrt(deg)
        h = jnp.dot(x_ref[...], w_ref[...], preferred_element_type=F32)
        hs = h * dv
        dinv_ref[...] = dv
        out_ref[0] = hs[:, :16]
        out_ref[1] = hs[:, 16:]

    return pl.pallas_call(
        body,
        grid=GRID,
        in_specs=[_row_spec(64), _row_spec(1), _row_spec(1), _full_spec(64, 32)],
        out_specs=[_row_spec(1), _pair_spec(16)],
        out_shape=[
            jax.ShapeDtypeStruct((NPAD, 1), F32),
            jax.ShapeDtypeStruct((2, NPAD, 16), F32),
        ],
    )(x_pad, deg0, deg1, W1)


def _tc_mid(agg, dinv, bias, W, Hin, Hout):
    def body(a_ref, dv_ref, b_ref, w_ref, out_ref):
        dv = dv_ref[...]
        cat = jnp.concatenate([a_ref[0], a_ref[1]], axis=1)
        z = jnp.maximum(dv * cat + b_ref[...], 0.0)
        h = jnp.dot(z, w_ref[...], preferred_element_type=F32)
        hs = h * dv
        out_ref[0] = hs[:, :Hout // 2]
        out_ref[1] = hs[:, Hout // 2:]

    return pl.pallas_call(
        body,
        grid=GRID,
        in_specs=[_pair_spec(Hin // 2), _row_spec(1), _full_spec(1, Hin),
                  _full_spec(Hin, Hout)],
        out_specs=[_pair_spec(Hout // 2)],
        out_shape=[jax.ShapeDtypeStruct((2, NPAD, Hout // 2), F32)],
    )(agg, dinv, bias, W)[0]


def _tc_final(agg, dinv, bias):
    def body(a_ref, dv_ref, b_ref, out_ref):
        cat = jnp.concatenate([a_ref[0], a_ref[1]], axis=1)
        emb = dv_ref[...] * cat + b_ref[...]
        nrm = jnp.sqrt(jnp.sum(emb * emb, axis=-1, keepdims=True))
        out_ref[...] = emb / jnp.maximum(nrm, 1e-12)

    return pl.pallas_call(
        body,
        grid=GRID,
        in_specs=[_pair_spec(32), _row_spec(1), _full_spec(1, 64)],
        out_specs=pl.BlockSpec((BR, 64), lambda i: (i, 0)),
        out_shape=jax.ShapeDtypeStruct((NPAD, 64), F32),
    )(agg, dinv, bias)


def kernel(x, edge_index, W1, b1, W2, b2, W3, b3):
    src = edge_index[0].astype(jnp.int32)
    dst = edge_index[1].astype(jnp.int32)
    pad_e = EPAD - E
    src2d = jnp.concatenate(
        [src, jnp.zeros((pad_e,), jnp.int32)]).reshape(IDX_ROWS, 128)
    dst2d = jnp.concatenate(
        [dst, jnp.full((pad_e,), NPAD - 1, jnp.int32)]).reshape(IDX_ROWS, 128)
    edges3d = jnp.stack([src2d, dst2d], axis=1)
    e4d_a = jnp.stack([src2d.reshape(-1, 1600),
                       dst2d.reshape(-1, 1600)], axis=1)
    e4d_b = jnp.stack([src2d.reshape(-1, 400),
                       dst2d.reshape(-1, 400)], axis=1)
    x_pad = jnp.zeros((NPAD, 64), F32).at[:N].set(x)
    zeros_col = jnp.zeros((NPAD, 1), F32)
    ones_rows = jnp.ones((128, 1), F32)

    deg_parts = _deg_call(edges3d, zeros_col, ones_rows)
    dinv, hs1 = _tc_prep(x_pad, deg_parts[0], deg_parts[1], W1)
    agg1 = _agg_call(hs1, e4d_a, 16, 1600)
    hs2 = _tc_mid(agg1, dinv, b1.reshape(1, 32), W2, 32, 64)
    agg2 = _agg_call(hs2, e4d_b, 32, 400)
    hs3 = _tc_mid(agg2, dinv, b2.reshape(1, 64), W3, 64, 64)
    agg3 = _agg_call(hs3, e4d_b, 32, 400)
    emb = _tc_final(agg3, dinv, b3.reshape(1, 64))
    return emb[:N]


# speedup vs baseline: 20.2454x; 1.2087x over previous
import functools

import jax
import jax.numpy as jnp
from jax import lax
from jax.experimental import pallas as pl
from jax.experimental.pallas import tpu as pltpu
from jax.experimental.pallas import tpu_sc as plsc

N = 50000
NPAD = 50176
E = 800000
EPAD = 819200
IDX_ROWS = EPAD // 128
NC, NS = 2, 16
TILE_ROWS = NPAD // NS
F32 = jnp.float32


def _sc_mesh():
    return plsc.VectorSubcoreMesh(core_axis_name="c", subcore_axis_name="s")


_SC_PARAMS = pltpu.CompilerParams(use_tc_tiling_on_sc=False)


def _init_chunk_for(ch):
    cap = ch * 128
    for c in (784, 448, 392, 224, 112, 56):
        if c <= cap:
            return c
    raise ValueError(ch)


def _deg_call(edge_idx, zeros_col, ones_rows):
    chunk_e = 1000
    edges_per_tile = E // (NC * NS)
    n_chunks = edges_per_tile // chunk_e

    @functools.partial(
        pl.kernel,
        out_type=jax.ShapeDtypeStruct((NC, NPAD, 1), F32),
        mesh=_sc_mesh(),
        compiler_params=_SC_PARAMS,
        scratch_types=[
            pltpu.VMEM((2, chunk_e), jnp.int32),
            pltpu.VMEM((chunk_e, 1), F32),
            pltpu.VMEM((TILE_ROWS, 1), F32),
            pltpu.VMEM_SHARED((NPAD, 1), F32),
            pltpu.SemaphoreType.DMA,
        ],
    )
    def deg_kernel(e_hbm, zero_hbm, ones_hbm, out_hbm,
                   dst_v, ones_v, stage_v, acc, sem):
        cid = lax.axis_index("c")
        sid = lax.axis_index("s")
        wid = cid * NS + sid
        base = sid * TILE_ROWS
        pltpu.sync_copy(zero_hbm.at[pl.ds(base, TILE_ROWS)], stage_v)
        pltpu.sync_copy(stage_v, acc.at[pl.ds(base, TILE_ROWS)])
        pltpu.sync_copy(ones_hbm, ones_v)
        plsc.subcore_barrier()

        e0 = wid * edges_per_tile

        def body(i, carry):
            off = e0 + i * chunk_e
            pltpu.sync_copy(e_hbm.at[1].at[pl.ds(off, chunk_e)], dst_v.at[0])
            pltpu.sync_copy(ones_v, acc.at[dst_v.at[0]], add=True)
            return carry

        lax.fori_loop(0, n_chunks, body, 0)
        plsc.subcore_barrier()
        pltpu.sync_copy(acc.at[pl.ds(base, TILE_ROWS)], stage_v)
        pltpu.sync_copy(stage_v, out_hbm.at[cid].at[pl.ds(base, TILE_ROWS)])

    return deg_kernel(edge_idx, zeros_col, ones_rows)


def _agg_call(table_pair, edge_idx, H, chunk_e):
    edges_per_tile = E // NS
    chunks_per_tile = edges_per_tile // chunk_e
    n_pairs = chunks_per_tile // 2

    @functools.partial(
        pl.kernel,
        out_type=jax.ShapeDtypeStruct((NC, NPAD, H), F32),
        mesh=_sc_mesh(),
        compiler_params=_SC_PARAMS,
        scratch_types=[
            pltpu.VMEM((2, chunk_e), jnp.int32),
            pltpu.VMEM((2, chunk_e), jnp.int32),
            pltpu.VMEM((chunk_e, H), F32),
            pltpu.VMEM((chunk_e, H), F32),
            pltpu.VMEM_SHARED((NPAD, H), F32),
            pltpu.SemaphoreType.DMA,
            pltpu.SemaphoreType.DMA,
            pltpu.SemaphoreType.DMA,
            pltpu.SemaphoreType.DMA,
        ],
    )
    def agg_kernel(table_hbm, e_hbm, out_hbm,
                   i0, i1, r0, r1, acc, g0, g1, s0, s1):
        cid = lax.axis_index("c")
        sid = lax.axis_index("s")
        base = sid * TILE_ROWS
        e0 = sid * edges_per_tile

        pltpu.sync_copy(table_hbm.at[cid].at[pl.ds(base, TILE_ROWS)],
                        acc.at[pl.ds(base, TILE_ROWS)])
        plsc.subcore_barrier()

        idx = (i0, i1)
        rows = (r0, r1)
        gsem = (g0, g1)
        ssem = (s0, s1)

        def load_i(b, c):
            off = e0 + c * chunk_e
            pltpu.sync_copy(e_hbm.at[0].at[pl.ds(off, chunk_e)], idx[b].at[0])
            pltpu.sync_copy(e_hbm.at[1].at[pl.ds(off, chunk_e)], idx[b].at[1])

        def fire_g(b):
            pltpu.async_copy(table_hbm.at[cid].at[idx[b].at[0]],
                             rows[b], gsem[b])

        def drain_g(b):
            pltpu.make_async_copy(table_hbm.at[cid].at[idx[b].at[0]],
                                  rows[b], gsem[b]).wait()

        def fire_s(b):
            pltpu.async_copy(rows[b], acc.at[idx[b].at[1]], ssem[b],
                             add=True)

        def drain_s(b):
            pltpu.make_async_copy(rows[b], acc.at[idx[b].at[1]],
                                  ssem[b]).wait()

        load_i(0, 0)
        fire_g(0)

        def body(p, carry):
            c = 2 * p
            load_i(1, c + 1)
            drain_g(0)
            fire_s(0)
            fire_g(1)
            drain_s(0)

            @pl.when(p + 1 < n_pairs)
            def _():
                load_i(0, c + 2)

            drain_g(1)
            fire_s(1)

            @pl.when(p + 1 < n_pairs)
            def _():
                fire_g(0)

            drain_s(1)
            return carry

        lax.fori_loop(0, n_pairs, body, 0)
        plsc.subcore_barrier()

        pltpu.sync_copy(acc.at[pl.ds(base, TILE_ROWS)],
                        out_hbm.at[cid].at[pl.ds(base, TILE_ROWS)])

    return agg_kernel(table_pair, edge_idx)


BR = 512
GRID = (NPAD // BR,)


def _row_spec(w):
    return pl.BlockSpec((BR, w), lambda i: (i, 0))


def _pair_spec(w):
    return pl.BlockSpec((2, BR, w), lambda i: (0, i, 0))


def _full_spec(a, b):
    return pl.BlockSpec((a, b), lambda i: (0, 0))


def _tc_prep(x_pad, deg0, deg1, W1):
    def body(x_ref, d0_ref, d1_ref, w_ref, dinv_ref, out_ref):
        deg = d0_ref[...] + d1_ref[...] + 1.0
        dv = lax.rsqrt(deg)
        h = jnp.dot(x_ref[...], w_ref[...], preferred_element_type=F32)
        hs = h * dv
        dinv_ref[...] = dv
        out_ref[0] = hs[:, :16]
        out_ref[1] = hs[:, 16:]

    return pl.pallas_call(
        body,
        grid=GRID,
        in_specs=[_row_spec(64), _row_spec(1), _row_spec(1), _full_spec(64, 32)],
        out_specs=[_row_spec(1), _pair_spec(16)],
        out_shape=[
            jax.ShapeDtypeStruct((NPAD, 1), F32),
            jax.ShapeDtypeStruct((2, NPAD, 16), F32),
        ],
    )(x_pad, deg0, deg1, W1)


def _tc_mid(agg, dinv, bias, W, Hin, Hout):
    def body(a_ref, dv_ref, b_ref, w_ref, out_ref):
        dv = dv_ref[...]
        cat = jnp.concatenate([a_ref[0], a_ref[1]], axis=1)
        z = jnp.maximum(dv * cat + b_ref[...], 0.0)
        h = jnp.dot(z, w_ref[...], preferred_element_type=F32)
        hs = h * dv
        out_ref[0] = hs[:, :Hout // 2]
        out_ref[1] = hs[:, Hout // 2:]

    return pl.pallas_call(
        body,
        grid=GRID,
        in_specs=[_pair_spec(Hin // 2), _row_spec(1), _full_spec(1, Hin),
                  _full_spec(Hin, Hout)],
        out_specs=[_pair_spec(Hout // 2)],
        out_shape=[jax.ShapeDtypeStruct((2, NPAD, Hout // 2), F32)],
    )(agg, dinv, bias, W)[0]


def _tc_final(agg, dinv, bias):
    def body(a_ref, dv_ref, b_ref, out_ref):
        cat = jnp.concatenate([a_ref[0], a_ref[1]], axis=1)
        emb = dv_ref[...] * cat + b_ref[...]
        nrm = jnp.sqrt(jnp.sum(emb * emb, axis=-1, keepdims=True))
        out_ref[...] = emb / jnp.maximum(nrm, 1e-12)

    return pl.pallas_call(
        body,
        grid=GRID,
        in_specs=[_pair_spec(32), _row_spec(1), _full_spec(1, 64)],
        out_specs=pl.BlockSpec((BR, 64), lambda i: (i, 0)),
        out_shape=jax.ShapeDtypeStruct((NPAD, 64), F32),
    )(agg, dinv, bias)


def kernel(x, edge_index, W1, b1, W2, b2, W3, b3):
    eidx = edge_index.astype(jnp.int32)
    x_pad = jnp.zeros((NPAD, 64), F32).at[:N].set(x)
    zeros_col = jnp.zeros((NPAD, 1), F32)
    ones_rows = jnp.ones((1000, 1), F32)

    deg_parts = _deg_call(eidx, zeros_col, ones_rows)
    dinv, hs1 = _tc_prep(x_pad, deg_parts[0], deg_parts[1], W1)
    agg1 = _agg_call(hs1, eidx, 16, 1000)
    hs2 = _tc_mid(agg1, dinv, b1.reshape(1, 32), W2, 32, 64)
    agg2 = _agg_call(hs2, eidx, 32, 200)
    hs3 = _tc_mid(agg2, dinv, b2.reshape(1, 64), W3, 64, 64)
    agg3 = _agg_call(hs3, eidx, 32, 200)
    emb = _tc_final(agg3, dinv, b3.reshape(1, 64))
    return emb[:N]


# speedup vs baseline: 22.0891x; 1.0911x over previous
import functools

import jax
import jax.numpy as jnp
from jax import lax
from jax.experimental import pallas as pl
from jax.experimental.pallas import tpu as pltpu
from jax.experimental.pallas import tpu_sc as plsc

N = 50000
NPAD = 50176
E = 800000
EPAD = 819200
IDX_ROWS = EPAD // 128
NC, NS = 2, 16
TILE_ROWS = NPAD // NS
F32 = jnp.float32


def _sc_mesh():
    return plsc.VectorSubcoreMesh(core_axis_name="c", subcore_axis_name="s")


_SC_PARAMS = pltpu.CompilerParams(use_tc_tiling_on_sc=False)


def _init_chunk_for(ch):
    cap = ch * 128
    for c in (784, 448, 392, 224, 112, 56):
        if c <= cap:
            return c
    raise ValueError(ch)


def _deg_call(edge_idx, zeros_col, ones_rows):
    chunk_e = 1000
    edges_per_tile = E // (NC * NS)
    n_chunks = edges_per_tile // chunk_e

    @functools.partial(
        pl.kernel,
        out_type=jax.ShapeDtypeStruct((NC, NPAD, 1), F32),
        mesh=_sc_mesh(),
        compiler_params=_SC_PARAMS,
        scratch_types=[
            pltpu.VMEM((2, chunk_e), jnp.int32),
            pltpu.VMEM((chunk_e, 1), F32),
            pltpu.VMEM((TILE_ROWS, 1), F32),
            pltpu.VMEM_SHARED((NPAD, 1), F32),
            pltpu.SemaphoreType.DMA,
        ],
    )
    def deg_kernel(e_hbm, zero_hbm, ones_hbm, out_hbm,
                   dst_v, ones_v, stage_v, acc, sem):
        cid = lax.axis_index("c")
        sid = lax.axis_index("s")
        wid = cid * NS + sid
        base = sid * TILE_ROWS
        pltpu.sync_copy(zero_hbm.at[pl.ds(base, TILE_ROWS)], stage_v)
        pltpu.sync_copy(stage_v, acc.at[pl.ds(base, TILE_ROWS)])
        pltpu.sync_copy(ones_hbm, ones_v)
        plsc.subcore_barrier()

        e0 = wid * edges_per_tile

        def body(i, carry):
            off = e0 + i * chunk_e
            pltpu.sync_copy(e_hbm.at[1].at[pl.ds(off, chunk_e)], dst_v.at[0])
            pltpu.sync_copy(ones_v, acc.at[dst_v.at[0]], add=True)
            return carry

        lax.fori_loop(0, n_chunks, body, 0)
        plsc.subcore_barrier()
        pltpu.sync_copy(acc.at[pl.ds(base, TILE_ROWS)], stage_v)
        pltpu.sync_copy(stage_v, out_hbm.at[cid].at[pl.ds(base, TILE_ROWS)])

    return deg_kernel(edge_idx, zeros_col, ones_rows)


def _agg_call(table_pair, edge_idx, H, chunk_e):
    edges_per_tile = E // NS
    chunks_per_tile = edges_per_tile // chunk_e
    n_pairs = chunks_per_tile // 2

    @functools.partial(
        pl.kernel,
        out_type=jax.ShapeDtypeStruct((NC, NPAD, H), F32),
        mesh=_sc_mesh(),
        compiler_params=_SC_PARAMS,
        scratch_types=[
            pltpu.VMEM((2, chunk_e), jnp.int32),
            pltpu.VMEM((2, chunk_e), jnp.int32),
            pltpu.VMEM((chunk_e, H), F32),
            pltpu.VMEM((chunk_e, H), F32),
            pltpu.VMEM_SHARED((NPAD, H), F32),
            pltpu.SemaphoreType.DMA,
            pltpu.SemaphoreType.DMA,
            pltpu.SemaphoreType.DMA,
            pltpu.SemaphoreType.DMA,
        ],
    )
    def agg_kernel(table_hbm, e_hbm, out_hbm,
                   i0, i1, r0, r1, acc, g0, g1, s0, s1):
        cid = lax.axis_index("c")
        sid = lax.axis_index("s")
        base = sid * TILE_ROWS
        e0 = sid * edges_per_tile

        pltpu.sync_copy(table_hbm.at[cid].at[pl.ds(base, TILE_ROWS)],
                        acc.at[pl.ds(base, TILE_ROWS)])
        plsc.subcore_barrier()

        idx = (i0, i1)
        rows = (r0, r1)
        gsem = (g0, g1)
        ssem = (s0, s1)

        def load_i(b, c):
            off = e0 + c * chunk_e
            pltpu.sync_copy(e_hbm.at[0].at[pl.ds(off, chunk_e)], idx[b].at[0])
            pltpu.sync_copy(e_hbm.at[1].at[pl.ds(off, chunk_e)], idx[b].at[1])

        def fire_g(b):
            pltpu.async_copy(table_hbm.at[cid].at[idx[b].at[0]],
                             rows[b], gsem[b])

        def drain_g(b):
            pltpu.make_async_copy(table_hbm.at[cid].at[idx[b].at[0]],
                                  rows[b], gsem[b]).wait()

        def fire_s(b):
            pltpu.async_copy(rows[b], acc.at[idx[b].at[1]], ssem[b],
                             add=True)

        def drain_s(b):
            pltpu.make_async_copy(rows[b], acc.at[idx[b].at[1]],
                                  ssem[b]).wait()

        load_i(0, 0)
        fire_g(0)

        def body(p, carry):
            c = 2 * p
            load_i(1, c + 1)
            drain_g(0)
            fire_s(0)
            fire_g(1)
            drain_s(0)

            @pl.when(p + 1 < n_pairs)
            def _():
                load_i(0, c + 2)

            drain_g(1)
            fire_s(1)

            @pl.when(p + 1 < n_pairs)
            def _():
                fire_g(0)

            drain_s(1)
            return carry

        lax.fori_loop(0, n_pairs, body, 0)
        plsc.subcore_barrier()

        pltpu.sync_copy(acc.at[pl.ds(base, TILE_ROWS)],
                        out_hbm.at[cid].at[pl.ds(base, TILE_ROWS)])

    return agg_kernel(table_pair, edge_idx)


BR = 512
GRID = (NPAD // BR,)


def _row_spec(w):
    return pl.BlockSpec((BR, w), lambda i: (i, 0))


def _pair_spec(w):
    return pl.BlockSpec((2, BR, w), lambda i: (0, i, 0))


def _full_spec(a, b):
    return pl.BlockSpec((a, b), lambda i: (0, 0))


def _tc_prep(x_pad, deg0, deg1, W1):
    def body(x_ref, d0_ref, d1_ref, w_ref, dinv_ref, out_ref):
        deg = d0_ref[...] + d1_ref[...] + 1.0
        dv = lax.rsqrt(deg)
        h = jnp.dot(x_ref[...], w_ref[...], preferred_element_type=F32)
        hs = h * dv
        dinv_ref[...] = dv
        out_ref[0] = hs[:, :16]
        out_ref[1] = hs[:, 16:]

    return pl.pallas_call(
        body,
        grid=GRID,
        in_specs=[_row_spec(64), _row_spec(1), _row_spec(1), _full_spec(64, 32)],
        out_specs=[_row_spec(1), _pair_spec(16)],
        out_shape=[
            jax.ShapeDtypeStruct((NPAD, 1), F32),
            jax.ShapeDtypeStruct((2, NPAD, 16), F32),
        ],
    )(x_pad, deg0, deg1, W1)


BRN = 3584
GRIDN = (NPAD // BRN,)


def _tc_mid_t(aggT, deg0T, deg1T, biasT, WT_a, WT_b, Hin, Hout):
    def body(a_ref, d0_ref, d1_ref, b_ref, wa_ref, wb_ref, out_ref):
        dv = lax.rsqrt(d0_ref[...] + d1_ref[...] + 1.0)
        cat = jnp.concatenate([a_ref[0], a_ref[1]], axis=0)
        z = jnp.maximum(dv * cat + b_ref[...], 0.0)
        ha = jnp.dot(wa_ref[...], z, preferred_element_type=F32) * dv
        hb = jnp.dot(wb_ref[...], z, preferred_element_type=F32) * dv
        out_ref[0] = ha
        out_ref[1] = hb

    return pl.pallas_call(
        body,
        grid=GRIDN,
        in_specs=[
            pl.BlockSpec((2, Hin // 2, BRN), lambda i: (0, 0, i)),
            pl.BlockSpec((1, BRN), lambda i: (0, i)),
            pl.BlockSpec((1, BRN), lambda i: (0, i)),
            pl.BlockSpec((Hin, 1), lambda i: (0, 0)),
            pl.BlockSpec((Hout // 2, Hin), lambda i: (0, 0)),
            pl.BlockSpec((Hout // 2, Hin), lambda i: (0, 0)),
        ],
        out_specs=[pl.BlockSpec((2, Hout // 2, BRN), lambda i: (0, 0, i))],
        out_shape=[jax.ShapeDtypeStruct((2, Hout // 2, NPAD), F32)],
    )(aggT, deg0T, deg1T, biasT, WT_a, WT_b)[0]


def _tc_final_t(aggT, deg0T, deg1T, biasT):
    def body(a_ref, d0_ref, d1_ref, b_ref, out_ref):
        dv = lax.rsqrt(d0_ref[...] + d1_ref[...] + 1.0)
        cat = jnp.concatenate([a_ref[0], a_ref[1]], axis=0)
        emb = dv * cat + b_ref[...]
        nrm = jnp.sqrt(jnp.sum(emb * emb, axis=0, keepdims=True))
        out_ref[...] = emb / jnp.maximum(nrm, 1e-12)

    return pl.pallas_call(
        body,
        grid=GRIDN,
        in_specs=[
            pl.BlockSpec((2, 32, BRN), lambda i: (0, 0, i)),
            pl.BlockSpec((1, BRN), lambda i: (0, i)),
            pl.BlockSpec((1, BRN), lambda i: (0, i)),
            pl.BlockSpec((64, 1), lambda i: (0, 0)),
        ],
        out_specs=pl.BlockSpec((64, BRN), lambda i: (0, i)),
        out_shape=jax.ShapeDtypeStruct((64, NPAD), F32),
    )(aggT, deg0T, deg1T, biasT)


def kernel(x, edge_index, W1, b1, W2, b2, W3, b3):
    eidx = edge_index.astype(jnp.int32)
    x_pad = jnp.zeros((NPAD, 64), F32).at[:N].set(x)
    zeros_col = jnp.zeros((NPAD, 1), F32)
    ones_rows = jnp.ones((1000, 1), F32)

    deg_parts = _deg_call(eidx, zeros_col, ones_rows)
    d0T = deg_parts[0].reshape(1, NPAD)
    d1T = deg_parts[1].reshape(1, NPAD)
    dinv, hs1 = _tc_prep(x_pad, deg_parts[0], deg_parts[1], W1)
    agg1 = _agg_call(hs1, eidx, 16, 1000)
    hs2T = _tc_mid_t(jnp.transpose(agg1, (0, 2, 1)), d0T, d1T,
                     b1.reshape(32, 1), W2[:, :32].T, W2[:, 32:].T, 32, 64)
    hs2 = jnp.transpose(hs2T, (0, 2, 1))
    agg2 = _agg_call(hs2, eidx, 32, 200)
    hs3T = _tc_mid_t(jnp.transpose(agg2, (0, 2, 1)), d0T, d1T,
                     b2.reshape(64, 1), W3[:, :32].T, W3[:, 32:].T, 64, 64)
    hs3 = jnp.transpose(hs3T, (0, 2, 1))
    agg3 = _agg_call(hs3, eidx, 32, 200)
    embT = _tc_final_t(jnp.transpose(agg3, (0, 2, 1)), d0T, d1T,
                       b3.reshape(64, 1))
    return embT.T[:N]
